# baseline (speedup 1.0000x reference)
import jax
import jax.numpy as jnp
from jax.experimental import pallas as pl
from jax.experimental.pallas import tpu as pltpu

B = 16384
N = 64
D_NODE = 16
OBS = 64
HID = 64
ACT = 5

CH = 128
NBUF = 6
G = B // CH


def _body(aid_ref, obs_ref, nob_ref, oh_ref,
          W1_ref, b1_ref, W2_ref, b2_ref,
          Wm1_ref, bm1_ref, Wm2_ref, bm2_ref,
          Wa_ref, ba_ref, adj_hbm, out_ref, buf, wrow_ref, sem):
    i = pl.program_id(0)

    def start(c):
        slot = jax.lax.rem(c, NBUF)
        pltpu.make_async_copy(
            adj_hbm.at[pl.ds(c * CH, CH)], buf.at[slot], sem.at[slot]
        ).start()

    @pl.when(i == 0)
    def _():
        for k in range(NBUF - 1):
            start(k)

    @pl.when(i + NBUF - 1 < G)
    def _():
        start(i + NBUF - 1)

    slot = jax.lax.rem(i, NBUF)
    pltpu.make_async_copy(
        adj_hbm.at[pl.ds(i * CH, CH)], buf.at[slot], sem.at[slot]
    ).wait()

    adj = buf[slot]
    nob = nob_ref[...]
    obs = obs_ref[...]
    onehot = oh_ref[...]

    f32 = jnp.float32
    rowsum = jnp.sum(adj, axis=-1) + 1.0

    Y = jax.lax.dot_general(
        nob, W1_ref[...],
        dimension_numbers=(((2,), (0,)), ((), ())),
        preferred_element_type=f32) + b1_ref[...][None, :, :]

    AY = jax.lax.dot_general(
        adj, Y,
        dimension_numbers=(((2,), (1,)), ((0,), (0,))),
        preferred_element_type=f32)
    R = jnp.maximum(AY + Y, 0.0)

    for k in range(CH):
        a = aid_ref[0, 0, k]
        wrow_ref[k, :] = buf[slot, k, a, :]
    adjrow = wrow_ref[...]

    rs_a = jnp.sum(adjrow, axis=-1, keepdims=True) + 1.0
    w = (adjrow + onehot) / (rowsum * rs_a)

    v = jax.lax.dot_general(
        w.reshape(CH, 1, N), R,
        dimension_numbers=(((2,), (1,)), ((0,), (0,))),
        preferred_element_type=f32).reshape(CH, HID)

    h2 = jnp.maximum(
        jnp.dot(v, W2_ref[...], preferred_element_type=f32)
        + b2_ref[...], 0.0)

    x = jnp.dot(obs, Wm1_ref[0:OBS, :], preferred_element_type=f32)
    x = x + jnp.dot(h2, Wm1_ref[OBS:OBS + HID, :], preferred_element_type=f32)
    x = jnp.maximum(x + bm1_ref[...], 0.0)
    x = jnp.maximum(
        jnp.dot(x, Wm2_ref[...], preferred_element_type=f32) + bm2_ref[...],
        0.0)
    act = jnp.tanh(
        jnp.dot(x, Wa_ref[...], preferred_element_type=f32) + ba_ref[...])
    out_ref[...] = act


def kernel(obs, node_obs, adj, agent_id, W1, b1, W2, b2, Wm1, bm1, Wm2, bm2,
           Wa, ba):
    aid2 = agent_id.astype(jnp.int32).reshape(G, 1, CH)
    onehot = jax.nn.one_hot(agent_id, N, dtype=jnp.float32)
    b1r = b1.reshape(1, HID)
    b2r = b2.reshape(1, HID)
    bm1r = bm1.reshape(1, HID)
    bm2r = bm2.reshape(1, HID)
    bar = ba.reshape(1, ACT)

    full = lambda *shape: pl.BlockSpec(shape, lambda i: (0,) * len(shape))
    out = pl.pallas_call(
        _body,
        grid=(G,),
        in_specs=[
            pl.BlockSpec((1, 1, CH), lambda i: (i, 0, 0),
                         memory_space=pltpu.SMEM),
            pl.BlockSpec((CH, OBS), lambda i: (i, 0)),
            pl.BlockSpec((CH, N, D_NODE), lambda i: (i, 0, 0)),
            pl.BlockSpec((CH, N), lambda i: (i, 0)),
            full(D_NODE, HID), full(1, HID),
            full(HID, HID), full(1, HID),
            full(OBS + HID, HID), full(1, HID),
            full(HID, HID), full(1, HID),
            full(HID, ACT), full(1, ACT),
            pl.BlockSpec(memory_space=pl.ANY),
        ],
        out_specs=pl.BlockSpec((CH, ACT), lambda i: (i, 0)),
        out_shape=jax.ShapeDtypeStruct((B, ACT), jnp.float32),
        scratch_shapes=[
            pltpu.VMEM((NBUF, CH, N, N), jnp.float32),
            pltpu.VMEM((CH, N), jnp.float32),
            pltpu.SemaphoreType.DMA((NBUF,)),
        ],
    )(aid2, obs, node_obs, onehot, W1, b1r, W2, b2r, Wm1, bm1r,
      Wm2, bm2r, Wa, bar, adj)
    return out


# speedup vs baseline: 1.0338x; 1.0338x over previous
import jax
import jax.numpy as jnp
from jax.experimental import pallas as pl
from jax.experimental.pallas import tpu as pltpu

B = 16384
N = 64
D_NODE = 16
OBS = 64
HID = 64
ACT = 5

CH = 256
NBUF = 3
G = B // CH


def _body(aid_ref, obs_ref, nob_ref, oh_ref,
          W1_ref, b1_ref, W2_ref, b2_ref,
          Wm1_ref, bm1_ref, Wm2_ref, bm2_ref,
          Wa_ref, ba_ref, adj_hbm, out_ref, buf, wrow_ref, sem):
    i = pl.program_id(0)

    def start(c):
        slot = jax.lax.rem(c, NBUF)
        pltpu.make_async_copy(
            adj_hbm.at[pl.ds(c * CH, CH)], buf.at[slot], sem.at[slot]
        ).start()

    @pl.when(i == 0)
    def _():
        for k in range(NBUF - 1):
            start(k)

    @pl.when(i + NBUF - 1 < G)
    def _():
        start(i + NBUF - 1)

    slot = jax.lax.rem(i, NBUF)
    pltpu.make_async_copy(
        adj_hbm.at[pl.ds(i * CH, CH)], buf.at[slot], sem.at[slot]
    ).wait()

    adj = buf[slot]
    nob = nob_ref[...]
    obs = obs_ref[...]
    onehot = oh_ref[...]

    f32 = jnp.float32
    rowsum = jnp.sum(adj, axis=-1) + 1.0

    Y = jax.lax.dot_general(
        nob, W1_ref[...],
        dimension_numbers=(((2,), (0,)), ((), ())),
        preferred_element_type=f32) + b1_ref[...][None, :, :]

    AY = jax.lax.dot_general(
        adj, Y,
        dimension_numbers=(((2,), (1,)), ((0,), (0,))),
        preferred_element_type=f32)
    R = jnp.maximum(AY + Y, 0.0)

    for k in range(CH):
        a = aid_ref[0, 0, k]
        wrow_ref[k, :] = buf[slot, k, a, :]
    adjrow = wrow_ref[...]

    rs_a = jnp.sum(adjrow, axis=-1, keepdims=True) + 1.0
    w = (adjrow + onehot) / (rowsum * rs_a)

    v = jax.lax.dot_general(
        w.reshape(CH, 1, N), R,
        dimension_numbers=(((2,), (1,)), ((0,), (0,))),
        preferred_element_type=f32).reshape(CH, HID)

    h2 = jnp.maximum(
        jnp.dot(v, W2_ref[...], preferred_element_type=f32)
        + b2_ref[...], 0.0)

    x = jnp.dot(obs, Wm1_ref[0:OBS, :], preferred_element_type=f32)
    x = x + jnp.dot(h2, Wm1_ref[OBS:OBS + HID, :], preferred_element_type=f32)
    x = jnp.maximum(x + bm1_ref[...], 0.0)
    x = jnp.maximum(
        jnp.dot(x, Wm2_ref[...], preferred_element_type=f32) + bm2_ref[...],
        0.0)
    act = jnp.tanh(
        jnp.dot(x, Wa_ref[...], preferred_element_type=f32) + ba_ref[...])
    out_ref[...] = act


def kernel(obs, node_obs, adj, agent_id, W1, b1, W2, b2, Wm1, bm1, Wm2, bm2,
           Wa, ba):
    aid2 = agent_id.astype(jnp.int32).reshape(G, 1, CH)
    onehot = jax.nn.one_hot(agent_id, N, dtype=jnp.float32)
    b1r = b1.reshape(1, HID)
    b2r = b2.reshape(1, HID)
    bm1r = bm1.reshape(1, HID)
    bm2r = bm2.reshape(1, HID)
    bar = ba.reshape(1, ACT)

    full = lambda *shape: pl.BlockSpec(shape, lambda i: (0,) * len(shape))
    out = pl.pallas_call(
        _body,
        grid=(G,),
        in_specs=[
            pl.BlockSpec((1, 1, CH), lambda i: (i, 0, 0),
                         memory_space=pltpu.SMEM),
            pl.BlockSpec((CH, OBS), lambda i: (i, 0)),
            pl.BlockSpec((CH, N, D_NODE), lambda i: (i, 0, 0)),
            pl.BlockSpec((CH, N), lambda i: (i, 0)),
            full(D_NODE, HID), full(1, HID),
            full(HID, HID), full(1, HID),
            full(OBS + HID, HID), full(1, HID),
            full(HID, HID), full(1, HID),
            full(HID, ACT), full(1, ACT),
            pl.BlockSpec(memory_space=pl.ANY),
        ],
        out_specs=pl.BlockSpec((CH, ACT), lambda i: (i, 0)),
        out_shape=jax.ShapeDtypeStruct((B, ACT), jnp.float32),
        scratch_shapes=[
            pltpu.VMEM((NBUF, CH, N, N), jnp.float32),
            pltpu.VMEM((CH, N), jnp.float32),
            pltpu.SemaphoreType.DMA((NBUF,)),
        ],
    )(aid2, obs, node_obs, onehot, W1, b1r, W2, b2r, Wm1, bm1r,
      Wm2, bm2r, Wa, bar, adj)
    return out


# speedup vs baseline: 1.0427x; 1.0086x over previous
import jax
import jax.numpy as jnp
from jax.experimental import pallas as pl
from jax.experimental.pallas import tpu as pltpu

B = 16384
N = 64
D_NODE = 16
OBS = 64
HID = 64
ACT = 5

BB = 256


def _body(aid_ref, obs_ref, nob_ref, adj_ref, oh_ref,
          W1_ref, b1_ref, W2_ref, b2_ref,
          Wm1_ref, bm1_ref, Wm2_ref, bm2_ref,
          Wa_ref, ba_ref, out_ref, wrow_ref):
    adj = adj_ref[...]
    nob = nob_ref[...]
    obs = obs_ref[...]
    onehot = oh_ref[...]

    f32 = jnp.float32
    rowsum = jnp.sum(adj, axis=-1) + 1.0

    Y = jax.lax.dot_general(
        nob, W1_ref[...],
        dimension_numbers=(((2,), (0,)), ((), ())),
        preferred_element_type=f32) + b1_ref[...][None, :, :]

    AY = jax.lax.dot_general(
        adj, Y,
        dimension_numbers=(((2,), (1,)), ((0,), (0,))),
        preferred_element_type=f32)
    R = jnp.maximum(AY + Y, 0.0)

    for k in range(BB):
        a = aid_ref[0, 0, k]
        wrow_ref[k, :] = adj_ref[k, a, :]
    adjrow = wrow_ref[...]

    rs_a = jnp.sum(adjrow, axis=-1, keepdims=True) + 1.0
    w = (adjrow + onehot) / (rowsum * rs_a)

    v = jax.lax.dot_general(
        w.reshape(BB, 1, N), R,
        dimension_numbers=(((2,), (1,)), ((0,), (0,))),
        preferred_element_type=f32).reshape(BB, HID)

    h2 = jnp.maximum(
        jnp.dot(v, W2_ref[...], preferred_element_type=f32)
        + b2_ref[...], 0.0)

    x = jnp.dot(obs, Wm1_ref[0:OBS, :], preferred_element_type=f32)
    x = x + jnp.dot(h2, Wm1_ref[OBS:OBS + HID, :], preferred_element_type=f32)
    x = jnp.maximum(x + bm1_ref[...], 0.0)
    x = jnp.maximum(
        jnp.dot(x, Wm2_ref[...], preferred_element_type=f32) + bm2_ref[...],
        0.0)
    act = jnp.tanh(
        jnp.dot(x, Wa_ref[...], preferred_element_type=f32) + ba_ref[...])
    out_ref[...] = act


def kernel(obs, node_obs, adj, agent_id, W1, b1, W2, b2, Wm1, bm1, Wm2, bm2,
           Wa, ba):
    G = B // BB
    aid2 = agent_id.astype(jnp.int32).reshape(G, 1, BB)
    onehot = jax.nn.one_hot(agent_id, N, dtype=jnp.float32)
    b1r = b1.reshape(1, HID)
    b2r = b2.reshape(1, HID)
    bm1r = bm1.reshape(1, HID)
    bm2r = bm2.reshape(1, HID)
    bar = ba.reshape(1, ACT)

    full = lambda *shape: pl.BlockSpec(shape, lambda i: (0,) * len(shape))
    out = pl.pallas_call(
        _body,
        grid=(G,),
        in_specs=[
            pl.BlockSpec((1, 1, BB), lambda i: (i, 0, 0),
                         memory_space=pltpu.SMEM),
            pl.BlockSpec((BB, OBS), lambda i: (i, 0)),
            pl.BlockSpec((BB, N, D_NODE), lambda i: (i, 0, 0)),
            pl.BlockSpec((BB, N, N), lambda i: (i, 0, 0)),
            pl.BlockSpec((BB, N), lambda i: (i, 0)),
            full(D_NODE, HID), full(1, HID),
            full(HID, HID), full(1, HID),
            full(OBS + HID, HID), full(1, HID),
            full(HID, HID), full(1, HID),
            full(HID, ACT), full(1, ACT),
        ],
        out_specs=pl.BlockSpec((BB, ACT), lambda i: (i, 0)),
        out_shape=jax.ShapeDtypeStruct((B, ACT), jnp.float32),
        scratch_shapes=[pltpu.VMEM((BB, N), jnp.float32)],
        compiler_params=pltpu.CompilerParams(
            dimension_semantics=("parallel",)),
    )(aid2, obs, node_obs, adj, onehot, W1, b1r, W2, b2r, Wm1, bm1r,
      Wm2, bm2r, Wa, bar)
    return out
